# Initial kernel scaffold; baseline (speedup 1.0000x reference)
import functools
import math

import jax
import jax.numpy as jnp
from jax import lax
from jax.experimental import pallas as pl
from jax.experimental.pallas import tpu as pltpu
from jax.experimental.pallas import tpu_sc as plsc

V = 100000
N = 1024
PAD = 0
SMOOTHING = 0.1
CONF = 1.0 - SMOOTHING
EPS = SMOOTHING / (V - 2)
CROW = SMOOTHING * math.log(EPS) + CONF * math.log(CONF)

VB = 2048
GRID = (V + VB - 1) // VB

_NC = 2
_NS = 16
NW = _NC * _NS
BPW = N // NW
CH = V // 16


def _tc_body(tgt_ref, x_ref, out_ref):
    j = pl.program_id(0)
    m = (tgt_ref[...] != PAD).astype(jnp.float32)
    x = x_ref[...]
    col = lax.broadcasted_iota(jnp.int32, x.shape, 1) + j * VB
    xv = jnp.where(col < V, x, 0.0)

    @pl.when(j == 0)
    def _init():
        out_ref[0, 0] = EPS * jnp.sum(x[:, 0:1] * m)

    out_ref[0, 0] += -EPS * jnp.sum(xv * m)


@functools.partial(
    pl.kernel,
    out_type=jax.ShapeDtypeStruct((NW, 16), jnp.float32),
    mesh=plsc.VectorSubcoreMesh(core_axis_name="c", subcore_axis_name="s"),
    scratch_types=[
        pltpu.VMEM((BPW,), jnp.int32),
        pltpu.VMEM((BPW,), jnp.int32),
        pltpu.VMEM((BPW, 16), jnp.float32),
        pltpu.VMEM((16,), jnp.float32),
        pltpu.SemaphoreType.DMA,
    ],
)
def _sc_gather(tgt_hbm, xtab_hbm, out_hbm, tgt_v, idx_v, rows_v, acc_v, sem):
    wid = lax.axis_index("s") * _NC + lax.axis_index("c")
    base = wid * BPW
    pltpu.sync_copy(tgt_hbm.at[pl.ds(base, BPW)], tgt_v)
    for c in range(BPW // 16):
        t16 = tgt_v[pl.ds(c * 16, 16)]
        nn = base + c * 16 + lax.iota(jnp.int32, 16)
        idx_v[pl.ds(c * 16, 16)] = nn * CH + jnp.right_shift(t16, 4)
    pltpu.async_copy(xtab_hbm.at[idx_v], rows_v, sem).wait()
    acc = jnp.zeros((16,), jnp.float32)
    for c in range(BPW // 16):
        t16 = tgt_v[pl.ds(c * 16, 16)]
        lane = jnp.bitwise_and(t16, 15)
        row = c * 16 + lax.iota(jnp.int32, 16)
        g = plsc.load_gather(rows_v, [row, lane])
        acc = acc + jnp.where(t16 != PAD, (EPS - CONF) * g + CROW, 0.0)
    acc_v[...] = acc
    pltpu.sync_copy(acc_v, out_hbm.at[wid])


def kernel(x, target):
    tgt = target.astype(jnp.int32)
    tc_out = pl.pallas_call(
        _tc_body,
        grid=(GRID,),
        in_specs=[
            pl.BlockSpec((N, 1), lambda j: (0, 0)),
            pl.BlockSpec((N, VB), lambda j: (0, j)),
        ],
        out_specs=pl.BlockSpec((1, 1), lambda j: (0, 0), memory_space=pltpu.SMEM),
        out_shape=jax.ShapeDtypeStruct((1, 1), jnp.float32),
    )(tgt.reshape(N, 1), x)
    sc_out = _sc_gather(tgt, x.reshape(N * CH, 16))
    return tc_out[0, 0] + jnp.sum(sc_out)


# speedup vs baseline: 1.6603x; 1.6603x over previous
import functools
import math

import jax
import jax.numpy as jnp
from jax import lax
from jax.experimental import pallas as pl
from jax.experimental.pallas import tpu as pltpu
from jax.experimental.pallas import tpu_sc as plsc

V = 100000
N = 1024
PAD = 0
SMOOTHING = 0.1
CONF = 1.0 - SMOOTHING
EPS = SMOOTHING / (V - 2)
CROW = SMOOTHING * math.log(EPS) + CONF * math.log(CONF)

VB = 2048
GRID = (V + VB - 1) // VB

_NC = 2
_NS = 16
NW = _NC * _NS
BPW = N // NW
CH = V // 16


def _tc_body(tgt_ref, x_ref, out_ref):
    j = pl.program_id(0)
    m = jnp.minimum(tgt_ref[...], 1).astype(jnp.float32)
    x = x_ref[...]
    col = lax.broadcasted_iota(jnp.int32, x.shape, 1) + j * VB
    xv = jnp.where(col < V, x, 0.0)

    @pl.when(j == 0)
    def _init():
        out_ref[0, 0] = EPS * jnp.sum(x[:, 0:1] * m)

    out_ref[0, 0] += -EPS * jnp.sum(xv * m)


@functools.cache
def _make_sc_gather():
    return functools.partial(
        pl.kernel,
        out_type=jax.ShapeDtypeStruct((NW, 16), jnp.float32),
        mesh=plsc.VectorSubcoreMesh(core_axis_name="c", subcore_axis_name="s"),
        scratch_types=[
            pltpu.VMEM((BPW,), jnp.int32),
            pltpu.VMEM((BPW, 8, 128), jnp.float32),
            pltpu.VMEM((16,), jnp.float32),
            pltpu.SemaphoreType.DMA,
        ],
    )(_sc_body)


def _sc_body(tgt_hbm, x_hbm, out_hbm, tgt_v, blk_v, acc_v, sem):
    wid = lax.axis_index("s") * _NC + lax.axis_index("c")
    base = wid * BPW
    pltpu.sync_copy(tgt_hbm.at[pl.ds(base, BPW)], tgt_v)
    t16s = []
    copies = []
    for c in range(BPW // 16):
        t16 = tgt_v[pl.ds(c * 16, 16)]
        t16s.append(t16)
        for i in range(16):
            k = c * 16 + i
            ti = t16[i]
            nb = pl.multiple_of(base + (k & ~7), 8)
            cb = pl.multiple_of(jnp.bitwise_and(ti, jnp.int32(~127)), 128)
            cp = pltpu.make_async_copy(
                x_hbm.at[pl.ds(nb, 8), pl.ds(cb, 128)], blk_v.at[k], sem)
            cp.start()
            copies.append(cp)
    for cp in copies:
        cp.wait()
    acc = jnp.zeros((16,), jnp.float32)
    lanes = lax.iota(jnp.int32, 16)
    for c in range(BPW // 16):
        t16 = t16s[c]
        for i in range(16):
            k = c * 16 + i
            ti = t16[i]
            q = pl.multiple_of(jnp.bitwise_and(jnp.right_shift(ti, 4), 7) * 16, 16)
            chunk = blk_v[k, k & 7, pl.ds(q, 16)]
            oh = (1 - jnp.minimum(jnp.abs(lanes - jnp.bitwise_and(ti, 15)), 1)) * jnp.minimum(ti, 1)
            acc = acc + oh.astype(jnp.float32) * ((EPS - CONF) * chunk + CROW)
    acc_v[...] = acc
    pltpu.sync_copy(acc_v, out_hbm.at[wid])


def kernel(x, target):
    tgt = target.astype(jnp.int32)
    tc_out = pl.pallas_call(
        _tc_body,
        grid=(GRID,),
        in_specs=[
            pl.BlockSpec((N, 1), lambda j: (0, 0)),
            pl.BlockSpec((N, VB), lambda j: (0, j)),
        ],
        out_specs=pl.BlockSpec((1, 1), lambda j: (0, 0), memory_space=pltpu.SMEM),
        out_shape=jax.ShapeDtypeStruct((1, 1), jnp.float32),
    )(tgt.reshape(N, 1), x)
    sc_out = _make_sc_gather()(tgt, x)
    return tc_out[0, 0] + jnp.sum(sc_out)


# speedup vs baseline: 1.7957x; 1.0816x over previous
import functools
import math

import jax
import jax.numpy as jnp
from jax import lax
from jax.experimental import pallas as pl
from jax.experimental.pallas import tpu as pltpu
from jax.experimental.pallas import tpu_sc as plsc

V = 100000
N = 1024
PAD = 0
SMOOTHING = 0.1
CONF = 1.0 - SMOOTHING
EPS = SMOOTHING / (V - 2)
CROW = SMOOTHING * math.log(EPS) + CONF * math.log(CONF)

VB = 2048
GRID = (V + VB - 1) // VB

_NC = 2
_NS = 16
NW = _NC * _NS
BPW = N // NW
CH = V // 16


def _sum128(x):
    s = x[:, 0:128]
    for g in range(1, x.shape[1] // 128):
        s = s + x[:, g * 128:(g + 1) * 128]
    return s


def _tc_body(tgt_ref, x_ref, out_ref, acc_ref, s0_ref):
    j = pl.program_id(0)
    nlast = pl.num_programs(0) - 1

    @pl.when(j == 0)
    def _init():
        m = jnp.minimum(tgt_ref[...], 1).astype(jnp.float32)
        acc_ref[...] = jnp.zeros_like(acc_ref)
        s0_ref[0, 0] = jnp.sum(x_ref[:, 0:1] * m)

    @pl.when(j != nlast)
    def _main():
        acc_ref[...] += _sum128(x_ref[...])

    @pl.when(j == nlast)
    def _tail():
        x = x_ref[...]
        col = lax.broadcasted_iota(jnp.int32, x.shape, 1) + j * VB
        acc_ref[...] += _sum128(jnp.where(col < V, x, 0.0))
        m = jnp.minimum(tgt_ref[...], 1).astype(jnp.float32)
        out_ref[0, 0] = EPS * s0_ref[0, 0] - EPS * jnp.sum(acc_ref[...] * m)


@functools.cache
def _make_sc_gather():
    return functools.partial(
        pl.kernel,
        out_type=jax.ShapeDtypeStruct((NW, 16), jnp.float32),
        mesh=plsc.VectorSubcoreMesh(core_axis_name="c", subcore_axis_name="s"),
        scratch_types=[
            pltpu.VMEM((BPW,), jnp.int32),
            pltpu.VMEM((BPW, 8, 128), jnp.float32),
            pltpu.VMEM((16,), jnp.float32),
            pltpu.SemaphoreType.DMA,
        ],
    )(_sc_body)


def _sc_body(tgt_hbm, x_hbm, out_hbm, tgt_v, blk_v, acc_v, sem):
    wid = lax.axis_index("s") * _NC + lax.axis_index("c")
    base = wid * BPW
    pltpu.sync_copy(tgt_hbm.at[pl.ds(base, BPW)], tgt_v)
    t16s = []
    copies = []
    for c in range(BPW // 16):
        t16 = tgt_v[pl.ds(c * 16, 16)]
        t16s.append(t16)
        for i in range(16):
            k = c * 16 + i
            ti = t16[i]
            nb = pl.multiple_of(base + (k & ~7), 8)
            cb = pl.multiple_of(jnp.bitwise_and(ti, jnp.int32(~127)), 128)
            cp = pltpu.make_async_copy(
                x_hbm.at[pl.ds(nb, 8), pl.ds(cb, 128)], blk_v.at[k], sem)
            cp.start()
            copies.append(cp)
    for cp in copies:
        cp.wait()
    acc = jnp.zeros((16,), jnp.float32)
    lanes = lax.iota(jnp.int32, 16)
    for c in range(BPW // 16):
        t16 = t16s[c]
        for i in range(16):
            k = c * 16 + i
            ti = t16[i]
            q = pl.multiple_of(jnp.bitwise_and(jnp.right_shift(ti, 4), 7) * 16, 16)
            chunk = blk_v[k, k & 7, pl.ds(q, 16)]
            oh = (1 - jnp.minimum(jnp.abs(lanes - jnp.bitwise_and(ti, 15)), 1)) * jnp.minimum(ti, 1)
            acc = acc + oh.astype(jnp.float32) * ((EPS - CONF) * chunk + CROW)
    acc_v[...] = acc
    pltpu.sync_copy(acc_v, out_hbm.at[wid])


def kernel(x, target):
    tgt = target.astype(jnp.int32)
    tc_out = pl.pallas_call(
        _tc_body,
        grid=(GRID,),
        in_specs=[
            pl.BlockSpec((N, 1), lambda j: (0, 0)),
            pl.BlockSpec((N, VB), lambda j: (0, j)),
        ],
        out_specs=pl.BlockSpec((1, 1), lambda j: (0, 0), memory_space=pltpu.SMEM),
        out_shape=jax.ShapeDtypeStruct((1, 1), jnp.float32),
        scratch_shapes=[
            pltpu.VMEM((N, 128), jnp.float32),
            pltpu.SMEM((1, 1), jnp.float32),
        ],
    )(tgt.reshape(N, 1), x)
    sc_out = _make_sc_gather()(tgt, x)
    return tc_out[0, 0] + jnp.sum(sc_out)


# speedup vs baseline: 1.8002x; 1.0025x over previous
import functools
import math

import jax
import jax.numpy as jnp
from jax import lax
from jax.experimental import pallas as pl
from jax.experimental.pallas import tpu as pltpu
from jax.experimental.pallas import tpu_sc as plsc

V = 100000
N = 1024
PAD = 0
SMOOTHING = 0.1
CONF = 1.0 - SMOOTHING
EPS = SMOOTHING / (V - 2)
CROW = SMOOTHING * math.log(EPS) + CONF * math.log(CONF)

VB = 512
NS_TC = 4
GRID = (V + NS_TC * VB - 1) // (NS_TC * VB)

_NC = 2
_NS = 16
NW = _NC * _NS
BPW = N // NW
CH = V // 16


def _sum128(x):
    s = x[:, 0:128]
    for g in range(1, x.shape[1] // 128):
        s = s + x[:, g * 128:(g + 1) * 128]
    return s


def _tc_body(tgt_ref, xa, xb, xc, xd, out_ref, acc_ref, s0_ref):
    j = pl.program_id(0)
    nlast = pl.num_programs(0) - 1

    @pl.when(j == 0)
    def _init():
        m = jnp.minimum(tgt_ref[...], 1).astype(jnp.float32)
        acc_ref[...] = jnp.zeros_like(acc_ref)
        s0_ref[0, 0] = jnp.sum(xa[:, 0:1] * m)

    s = _sum128(xa[...]) + _sum128(xb[...]) + _sum128(xc[...])

    @pl.when(j != nlast)
    def _main():
        acc_ref[...] += s + _sum128(xd[...])

    @pl.when(j == nlast)
    def _tail():
        x = xd[...]
        col = lax.broadcasted_iota(jnp.int32, x.shape, 1) + (j + 3 * GRID) * VB
        acc_ref[...] += s + _sum128(jnp.where(col < V, x, 0.0))
        m = jnp.minimum(tgt_ref[...], 1).astype(jnp.float32)
        out_ref[0, 0] = EPS * s0_ref[0, 0] - EPS * jnp.sum(acc_ref[...] * m)


@functools.cache
def _make_sc_gather():
    return functools.partial(
        pl.kernel,
        out_type=jax.ShapeDtypeStruct((NW, 16), jnp.float32),
        mesh=plsc.VectorSubcoreMesh(core_axis_name="c", subcore_axis_name="s"),
        scratch_types=[
            pltpu.VMEM((BPW,), jnp.int32),
            pltpu.VMEM((BPW, 8, 128), jnp.float32),
            pltpu.VMEM((16,), jnp.float32),
            pltpu.SemaphoreType.DMA,
        ],
    )(_sc_body)


def _sc_body(tgt_hbm, x_hbm, out_hbm, tgt_v, blk_v, acc_v, sem):
    wid = lax.axis_index("s") * _NC + lax.axis_index("c")
    base = wid * BPW
    pltpu.sync_copy(tgt_hbm.at[pl.ds(base, BPW)], tgt_v)
    t16s = []
    copies = []
    for c in range(BPW // 16):
        t16 = tgt_v[pl.ds(c * 16, 16)]
        t16s.append(t16)
        for i in range(16):
            k = c * 16 + i
            ti = t16[i]
            nb = pl.multiple_of(base + (k & ~7), 8)
            cb = pl.multiple_of(jnp.bitwise_and(ti, jnp.int32(~127)), 128)
            cp = pltpu.make_async_copy(
                x_hbm.at[pl.ds(nb, 8), pl.ds(cb, 128)], blk_v.at[k], sem)
            cp.start()
            copies.append(cp)
    for cp in copies:
        cp.wait()
    acc = jnp.zeros((16,), jnp.float32)
    lanes = lax.iota(jnp.int32, 16)
    for c in range(BPW // 16):
        t16 = t16s[c]
        for i in range(16):
            k = c * 16 + i
            ti = t16[i]
            q = pl.multiple_of(jnp.bitwise_and(jnp.right_shift(ti, 4), 7) * 16, 16)
            chunk = blk_v[k, k & 7, pl.ds(q, 16)]
            oh = (1 - jnp.minimum(jnp.abs(lanes - jnp.bitwise_and(ti, 15)), 1)) * jnp.minimum(ti, 1)
            acc = acc + oh.astype(jnp.float32) * ((EPS - CONF) * chunk + CROW)
    acc_v[...] = acc
    pltpu.sync_copy(acc_v, out_hbm.at[wid])


def kernel(x, target):
    tgt = target.astype(jnp.int32)
    tc_out = pl.pallas_call(
        _tc_body,
        grid=(GRID,),
        in_specs=[
            pl.BlockSpec((N, 1), lambda j: (0, 0)),
            pl.BlockSpec((N, VB), lambda j: (0, j)),
            pl.BlockSpec((N, VB), lambda j: (0, j + GRID)),
            pl.BlockSpec((N, VB), lambda j: (0, j + 2 * GRID)),
            pl.BlockSpec((N, VB), lambda j: (0, j + 3 * GRID)),
        ],
        out_specs=pl.BlockSpec((1, 1), lambda j: (0, 0), memory_space=pltpu.SMEM),
        out_shape=jax.ShapeDtypeStruct((1, 1), jnp.float32),
        scratch_shapes=[
            pltpu.VMEM((N, 128), jnp.float32),
            pltpu.SMEM((1, 1), jnp.float32),
        ],
    )(tgt.reshape(N, 1), x, x, x, x)
    sc_out = _make_sc_gather()(tgt, x)
    return tc_out[0, 0] + jnp.sum(sc_out)
